# initial kernel scaffold (unmeasured)
import jax
import jax.numpy as jnp
from jax import lax
from jax.experimental import pallas as pl
from jax.experimental.pallas import tpu as pltpu

N_DEV = 8
M, N = 4096, 2048
CH = M // N_DEV


def _ar_body(p_ref, out_ref, comm_ref, send_sems, recv_sems, credit_sem):
    del p_ref
    my = lax.axis_index("i")
    left = jnp.mod(my - 1, N_DEV)
    right = jnp.mod(my + 1, N_DEV)

    barrier = pltpu.get_barrier_semaphore()
    for nbr in (left, right):
        pl.semaphore_signal(
            barrier, inc=1, device_id=(nbr,),
            device_id_type=pl.DeviceIdType.MESH,
        )
    pl.semaphore_wait(barrier, 2)

    for s in range(2 * N_DEV - 2):
        slot = s % 2
        send_c = jnp.mod(my - s, N_DEV)
        recv_c = jnp.mod(my - s - 1, N_DEV)

        if s >= 2:
            pl.semaphore_wait(credit_sem, 1)
        rdma = pltpu.make_async_remote_copy(
            src_ref=out_ref.at[pl.ds(send_c * CH, CH), :],
            dst_ref=comm_ref.at[slot],
            send_sem=send_sems.at[slot],
            recv_sem=recv_sems.at[slot],
            device_id=(right,),
            device_id_type=pl.DeviceIdType.MESH,
        )
        rdma.start()
        rdma.wait()

        rows = pl.ds(recv_c * CH, CH)
        if s < N_DEV - 1:
            out_ref[rows, :] += comm_ref[slot]
        else:
            out_ref[rows, :] = comm_ref[slot]
        pl.semaphore_signal(
            credit_sem, inc=1, device_id=(left,),
            device_id_type=pl.DeviceIdType.MESH,
        )

    pl.semaphore_wait(credit_sem, 2)


def kernel(x, w_mat, scale_x, scale_w):
    partial = lax.dot_general(
        x.astype(jnp.bfloat16),
        w_mat.astype(jnp.bfloat16),
        (((1,), (0,)), ((), ())),
        preferred_element_type=jnp.float32,
    )
    partial = partial * (scale_x[0] * scale_w[0])

    return pl.pallas_call(
        _ar_body,
        out_shape=jax.ShapeDtypeStruct((M, N), jnp.float32),
        in_specs=[pl.BlockSpec(memory_space=pltpu.VMEM)],
        out_specs=pl.BlockSpec(memory_space=pltpu.VMEM),
        scratch_shapes=[
            pltpu.VMEM((2, CH, N), jnp.float32),
            pltpu.SemaphoreType.DMA((2,)),
            pltpu.SemaphoreType.DMA((2,)),
            pltpu.SemaphoreType.REGULAR,
        ],
        input_output_aliases={0: 0},
        compiler_params=pltpu.CompilerParams(collective_id=0),
    )(partial)


# baseline (device time: 723778 ns/iter reference)
import jax
import jax.numpy as jnp
from jax import lax
from jax.experimental import pallas as pl
from jax.experimental.pallas import tpu as pltpu

N_DEV = 8
M, N = 4096, 2048
CH = M // N_DEV


def _ar_body(p_ref, out_ref, comm_ref, send_sems, recv_sems, credit_sem,
             copy_sem):
    local_copy = pltpu.make_async_copy(p_ref, out_ref, copy_sem)
    local_copy.start()
    my = lax.axis_index("i")
    left = jnp.mod(my - 1, N_DEV)
    right = jnp.mod(my + 1, N_DEV)

    barrier = pltpu.get_barrier_semaphore()
    for nbr in (left, right):
        pl.semaphore_signal(
            barrier, inc=1, device_id=(nbr,),
            device_id_type=pl.DeviceIdType.MESH,
        )
    pl.semaphore_wait(barrier, 2)
    local_copy.wait()

    for s in range(2 * N_DEV - 2):
        slot = s % 2
        send_c = jnp.mod(my - s, N_DEV)
        recv_c = jnp.mod(my - s - 1, N_DEV)

        if s >= 2:
            pl.semaphore_wait(credit_sem, 1)
        rdma = pltpu.make_async_remote_copy(
            src_ref=out_ref.at[pl.ds(send_c * CH, CH), :],
            dst_ref=comm_ref.at[slot],
            send_sem=send_sems.at[slot],
            recv_sem=recv_sems.at[slot],
            device_id=(right,),
            device_id_type=pl.DeviceIdType.MESH,
        )
        rdma.start()
        rdma.wait()

        rows = pl.ds(recv_c * CH, CH)
        if s < N_DEV - 1:
            out_ref[rows, :] += comm_ref[slot]
        else:
            out_ref[rows, :] = comm_ref[slot]
        pl.semaphore_signal(
            credit_sem, inc=1, device_id=(left,),
            device_id_type=pl.DeviceIdType.MESH,
        )

    pl.semaphore_wait(credit_sem, 2)


def kernel(x, w_mat, scale_x, scale_w):
    partial = lax.dot_general(
        x.astype(jnp.bfloat16),
        w_mat.astype(jnp.bfloat16),
        (((1,), (0,)), ((), ())),
        preferred_element_type=jnp.float32,
    )
    partial = partial * (scale_x[0] * scale_w[0])

    return pl.pallas_call(
        _ar_body,
        out_shape=jax.ShapeDtypeStruct((M, N), jnp.float32),
        in_specs=[pl.BlockSpec(memory_space=pl.MemorySpace.ANY)],
        out_specs=pl.BlockSpec(memory_space=pltpu.VMEM),
        scratch_shapes=[
            pltpu.VMEM((2, CH, N), jnp.float32),
            pltpu.SemaphoreType.DMA((2,)),
            pltpu.SemaphoreType.DMA((2,)),
            pltpu.SemaphoreType.REGULAR,
            pltpu.SemaphoreType.DMA,
        ],
        compiler_params=pltpu.CompilerParams(
            collective_id=0,
            vmem_limit_bytes=60 * 1024 * 1024,
        ),
    )(partial)


# device time: 414263 ns/iter; 1.7471x vs baseline; 1.7471x over previous
import jax
import jax.numpy as jnp
from jax import lax
from jax.experimental import pallas as pl
from jax.experimental.pallas import tpu as pltpu

N_DEV = 8
M, N = 4096, 2048
HALF = M // 2
CH = HALF // N_DEV


def _ar_body(p_ref, out_ref, comm_cw, comm_ccw, send_cw, recv_cw,
             send_ccw, recv_ccw, credit_cw, credit_ccw, copy_sem):
    local_copy = pltpu.make_async_copy(p_ref, out_ref, copy_sem)
    local_copy.start()
    my = lax.axis_index("i")
    left = jnp.mod(my - 1, N_DEV)
    right = jnp.mod(my + 1, N_DEV)

    barrier = pltpu.get_barrier_semaphore()
    for nbr in (left, right):
        pl.semaphore_signal(
            barrier, inc=1, device_id=(nbr,),
            device_id_type=pl.DeviceIdType.MESH,
        )
    pl.semaphore_wait(barrier, 2)
    local_copy.wait()

    def cw_rows(c):
        return pl.ds(c * CH, CH)

    def ccw_rows(c):
        return pl.ds(HALF + c * CH, CH)

    for s in range(2 * N_DEV - 2):
        slot = s % 2
        cs_cw = jnp.mod(my - s, N_DEV)
        cr_cw = jnp.mod(my - s - 1, N_DEV)
        cs_ccw = jnp.mod(my + s, N_DEV)
        cr_ccw = jnp.mod(my + s + 1, N_DEV)

        if s >= 2:
            pl.semaphore_wait(credit_cw, 1)
            pl.semaphore_wait(credit_ccw, 1)
        r_cw = pltpu.make_async_remote_copy(
            src_ref=out_ref.at[cw_rows(cs_cw), :],
            dst_ref=comm_cw.at[slot],
            send_sem=send_cw.at[slot],
            recv_sem=recv_cw.at[slot],
            device_id=(right,),
            device_id_type=pl.DeviceIdType.MESH,
        )
        r_ccw = pltpu.make_async_remote_copy(
            src_ref=out_ref.at[ccw_rows(cs_ccw), :],
            dst_ref=comm_ccw.at[slot],
            send_sem=send_ccw.at[slot],
            recv_sem=recv_ccw.at[slot],
            device_id=(left,),
            device_id_type=pl.DeviceIdType.MESH,
        )
        r_cw.start()
        r_ccw.start()

        r_cw.wait()
        rows = cw_rows(cr_cw)
        if s < N_DEV - 1:
            out_ref[rows, :] += comm_cw[slot]
        else:
            out_ref[rows, :] = comm_cw[slot]
        pl.semaphore_signal(
            credit_cw, inc=1, device_id=(left,),
            device_id_type=pl.DeviceIdType.MESH,
        )

        r_ccw.wait()
        rows = ccw_rows(cr_ccw)
        if s < N_DEV - 1:
            out_ref[rows, :] += comm_ccw[slot]
        else:
            out_ref[rows, :] = comm_ccw[slot]
        pl.semaphore_signal(
            credit_ccw, inc=1, device_id=(right,),
            device_id_type=pl.DeviceIdType.MESH,
        )

    pl.semaphore_wait(credit_cw, 2)
    pl.semaphore_wait(credit_ccw, 2)


def kernel(x, w_mat, scale_x, scale_w):
    partial = lax.dot_general(
        x.astype(jnp.bfloat16),
        w_mat.astype(jnp.bfloat16),
        (((1,), (0,)), ((), ())),
        preferred_element_type=jnp.float32,
    )
    partial = partial * (scale_x[0] * scale_w[0])

    return pl.pallas_call(
        _ar_body,
        out_shape=jax.ShapeDtypeStruct((M, N), jnp.float32),
        in_specs=[pl.BlockSpec(memory_space=pl.MemorySpace.ANY)],
        out_specs=pl.BlockSpec(memory_space=pltpu.VMEM),
        scratch_shapes=[
            pltpu.VMEM((2, CH, N), jnp.float32),
            pltpu.VMEM((2, CH, N), jnp.float32),
            pltpu.SemaphoreType.DMA((2,)),
            pltpu.SemaphoreType.DMA((2,)),
            pltpu.SemaphoreType.DMA((2,)),
            pltpu.SemaphoreType.DMA((2,)),
            pltpu.SemaphoreType.REGULAR,
            pltpu.SemaphoreType.REGULAR,
            pltpu.SemaphoreType.DMA,
        ],
        compiler_params=pltpu.CompilerParams(
            collective_id=0,
            vmem_limit_bytes=60 * 1024 * 1024,
        ),
    )(partial)


# device time: 293429 ns/iter; 2.4666x vs baseline; 1.4118x over previous
import jax
import jax.numpy as jnp
from jax import lax
from jax.experimental import pallas as pl
from jax.experimental.pallas import tpu as pltpu

N_DEV = 8
M, N = 4096, 2048
ENGINE_ROWS = (1408, 1344, 1344)
ENGINE_BASE = (0, 1408, 2752)
MASKS = ((1, 3, 4), (3, 4, 1), (4, 1, 3))
_MESH = pl.DeviceIdType.MESH


def _keep_bit(my, m):
    if m == 1:
        return (my ^ (my >> 1)) & 1
    if m == 3:
        return (my >> 1) & 1
    return (my >> 2) & 1


def _ar_body(p_ref, out_ref, buf0, buf1, buf2,
             rs_send, rs_recv, ag_send, ag_recv, credits, copy_sem):
    bufs = (buf0, buf1, buf2)
    local_copy = pltpu.make_async_copy(p_ref, out_ref, copy_sem)
    local_copy.start()
    my = lax.axis_index("i")

    barrier = pltpu.get_barrier_semaphore()
    for m in (1, 3, 4):
        pl.semaphore_signal(
            barrier, inc=1, device_id=(my ^ m,), device_id_type=_MESH,
        )
    pl.semaphore_wait(barrier, 3)
    local_copy.wait()

    S, KB = [], []
    for e in range(3):
        s, kb = [ENGINE_BASE[e]], []
        for r in range(3):
            b = _keep_bit(my, MASKS[e][r])
            kb.append(b)
            s.append(s[-1] + b * (ENGINE_ROWS[e] >> (r + 1)))
        S.append(s)
        KB.append(kb)

    for r in range(3):
        rdmas = []
        for e in range(3):
            H = ENGINE_ROWS[e] >> (r + 1)
            partner = my ^ MASKS[e][r]
            if r > 0:
                pl.semaphore_wait(credits.at[e], 1)
            send_base = S[e][r] + (1 - KB[e][r]) * H
            rd = pltpu.make_async_remote_copy(
                src_ref=out_ref.at[pl.ds(send_base, H), :],
                dst_ref=bufs[e].at[pl.ds(0, H), :],
                send_sem=rs_send.at[e, r],
                recv_sem=rs_recv.at[e, r],
                device_id=(partner,),
                device_id_type=_MESH,
            )
            rd.start()
            rdmas.append(rd)
        for e in range(3):
            H = ENGINE_ROWS[e] >> (r + 1)
            rdmas[e].wait_recv()
            rows = pl.ds(S[e][r + 1], H)
            out_ref[rows, :] += bufs[e][pl.ds(0, H), :]
            rdmas[e].wait_send()
            if r < 2:
                pl.semaphore_signal(
                    credits.at[e], inc=1,
                    device_id=(my ^ MASKS[e][r + 1],), device_id_type=_MESH,
                )

    for k in (2, 1, 0):
        rdmas = []
        for e in range(3):
            L = ENGINE_ROWS[e] >> (k + 1)
            rows = pl.ds(S[e][k + 1], L)
            rd = pltpu.make_async_remote_copy(
                src_ref=out_ref.at[rows, :],
                dst_ref=out_ref.at[rows, :],
                send_sem=ag_send.at[e, k],
                recv_sem=ag_recv.at[e, k],
                device_id=(my ^ MASKS[e][k],),
                device_id_type=_MESH,
            )
            rd.start()
            rdmas.append(rd)
        for e in range(3):
            rdmas[e].wait()


def kernel(x, w_mat, scale_x, scale_w):
    partial = lax.dot_general(
        x.astype(jnp.bfloat16),
        w_mat.astype(jnp.bfloat16),
        (((1,), (0,)), ((), ())),
        preferred_element_type=jnp.float32,
    )
    partial = partial * (scale_x[0] * scale_w[0])

    return pl.pallas_call(
        _ar_body,
        out_shape=jax.ShapeDtypeStruct((M, N), jnp.float32),
        in_specs=[pl.BlockSpec(memory_space=pl.MemorySpace.ANY)],
        out_specs=pl.BlockSpec(memory_space=pltpu.VMEM),
        scratch_shapes=[
            pltpu.VMEM((ENGINE_ROWS[0] // 2, N), jnp.float32),
            pltpu.VMEM((ENGINE_ROWS[1] // 2, N), jnp.float32),
            pltpu.VMEM((ENGINE_ROWS[2] // 2, N), jnp.float32),
            pltpu.SemaphoreType.DMA((3, 3)),
            pltpu.SemaphoreType.DMA((3, 3)),
            pltpu.SemaphoreType.DMA((3, 3)),
            pltpu.SemaphoreType.DMA((3, 3)),
            pltpu.SemaphoreType.REGULAR((3,)),
            pltpu.SemaphoreType.DMA,
        ],
        compiler_params=pltpu.CompilerParams(
            collective_id=0,
            vmem_limit_bytes=60 * 1024 * 1024,
        ),
    )(partial)


# device time: 285315 ns/iter; 2.5368x vs baseline; 1.0284x over previous
import jax
import jax.numpy as jnp
from jax import lax
from jax.experimental import pallas as pl
from jax.experimental.pallas import tpu as pltpu

N_DEV = 8
M, N = 4096, 2048
ENGINE_ROWS = (1408, 1344, 1344)
ENGINE_BASE = (0, 1408, 2752)
MASKS = ((1, 3, 4), (3, 4, 1), (4, 1, 3))
_MESH = pl.DeviceIdType.MESH


def _keep_bit(my, m):
    if m == 1:
        return (my ^ (my >> 1)) & 1
    if m == 3:
        return (my >> 1) & 1
    return (my >> 2) & 1


def _ar_body(p_ref, out_ref, buf0, buf1, buf2,
             rs_send, rs_recv, ag_send, ag_recv, credits, copy_sem):
    bufs = (buf0, buf1, buf2)
    local_copy = pltpu.make_async_copy(p_ref, out_ref, copy_sem)
    local_copy.start()
    my = lax.axis_index("i")

    barrier = pltpu.get_barrier_semaphore()
    for m in (1, 3, 4):
        pl.semaphore_signal(
            barrier, inc=1, device_id=(my ^ m,), device_id_type=_MESH,
        )
    pl.semaphore_wait(barrier, 3)

    S, KB = [], []
    for e in range(3):
        s, kb = [ENGINE_BASE[e]], []
        for r in range(3):
            b = _keep_bit(my, MASKS[e][r])
            kb.append(b)
            s.append(s[-1] + b * (ENGINE_ROWS[e] >> (r + 1)))
        S.append(s)
        KB.append(kb)

    for r in range(3):
        n_sub = 2 if r < 2 else 1
        rdmas = []
        for e in range(3):
            H = ENGINE_ROWS[e] >> (r + 1)
            Hs = H // n_sub
            partner = my ^ MASKS[e][r]
            if r > 0:
                pl.semaphore_wait(credits.at[e], 1)
            send_base = S[e][r] + (1 - KB[e][r]) * H
            src = p_ref if r == 0 else out_ref
            subs = []
            for j in range(n_sub):
                rd = pltpu.make_async_remote_copy(
                    src_ref=src.at[pl.ds(send_base + j * Hs, Hs), :],
                    dst_ref=bufs[e].at[pl.ds(j * Hs, Hs), :],
                    send_sem=rs_send.at[e, r, j],
                    recv_sem=rs_recv.at[e, r, j],
                    device_id=(partner,),
                    device_id_type=_MESH,
                )
                rd.start()
                subs.append(rd)
            rdmas.append(subs)
        if r == 0:
            local_copy.wait()
        for j in range(n_sub):
            for e in range(3):
                Hs = (ENGINE_ROWS[e] >> (r + 1)) // n_sub
                rdmas[e][j].wait_recv()
                rows = pl.ds(S[e][r + 1] + j * Hs, Hs)
                out_ref[rows, :] += bufs[e][pl.ds(j * Hs, Hs), :]
        for e in range(3):
            for j in range(n_sub):
                rdmas[e][j].wait_send()
            if r < 2:
                pl.semaphore_signal(
                    credits.at[e], inc=1,
                    device_id=(my ^ MASKS[e][r + 1],), device_id_type=_MESH,
                )

    for k in (2, 1, 0):
        rdmas = []
        for e in range(3):
            L = ENGINE_ROWS[e] >> (k + 1)
            rows = pl.ds(S[e][k + 1], L)
            rd = pltpu.make_async_remote_copy(
                src_ref=out_ref.at[rows, :],
                dst_ref=out_ref.at[rows, :],
                send_sem=ag_send.at[e, k],
                recv_sem=ag_recv.at[e, k],
                device_id=(my ^ MASKS[e][k],),
                device_id_type=_MESH,
            )
            rd.start()
            rdmas.append(rd)
        for e in range(3):
            rdmas[e].wait()


def kernel(x, w_mat, scale_x, scale_w):
    partial = lax.dot_general(
        x.astype(jnp.bfloat16),
        w_mat.astype(jnp.bfloat16),
        (((1,), (0,)), ((), ())),
        preferred_element_type=jnp.float32,
    )
    partial = partial * (scale_x[0] * scale_w[0])

    return pl.pallas_call(
        _ar_body,
        out_shape=jax.ShapeDtypeStruct((M, N), jnp.float32),
        in_specs=[pl.BlockSpec(memory_space=pl.MemorySpace.ANY)],
        out_specs=pl.BlockSpec(memory_space=pltpu.VMEM),
        scratch_shapes=[
            pltpu.VMEM((ENGINE_ROWS[0] // 2, N), jnp.float32),
            pltpu.VMEM((ENGINE_ROWS[1] // 2, N), jnp.float32),
            pltpu.VMEM((ENGINE_ROWS[2] // 2, N), jnp.float32),
            pltpu.SemaphoreType.DMA((3, 3, 2)),
            pltpu.SemaphoreType.DMA((3, 3, 2)),
            pltpu.SemaphoreType.DMA((3, 3)),
            pltpu.SemaphoreType.DMA((3, 3)),
            pltpu.SemaphoreType.REGULAR((3,)),
            pltpu.SemaphoreType.DMA,
        ],
        compiler_params=pltpu.CompilerParams(
            collective_id=0,
            vmem_limit_bytes=60 * 1024 * 1024,
        ),
    )(partial)


# device time: 276091 ns/iter; 2.6215x vs baseline; 1.0334x over previous
import jax
import jax.numpy as jnp
from jax import lax
from jax.experimental import pallas as pl
from jax.experimental.pallas import tpu as pltpu

N_DEV = 8
M, N, K = 4096, 2048, 512
ENGINE_ROWS = (1408, 1344, 1344)
ENGINE_BASE = (0, 1408, 2752)
MASKS = ((1, 3, 4), (3, 4, 1), (4, 1, 3))
_MESH = pl.DeviceIdType.MESH


def _keep_bit(my, m):
    if m == 1:
        return (my ^ (my >> 1)) & 1
    if m == 3:
        return (my >> 1) & 1
    return (my >> 2) & 1


def _ar_body(x_ref, w_ref, sx_ref, sw_ref, out_ref, buf0, buf1, buf2,
             rs_send, rs_recv, ag_send, ag_recv, credits):
    bufs = (buf0, buf1, buf2)
    my = lax.axis_index("i")

    barrier = pltpu.get_barrier_semaphore()
    for m in (1, 3, 4):
        pl.semaphore_signal(
            barrier, inc=1, device_id=(my ^ m,), device_id_type=_MESH,
        )
    pl.semaphore_wait(barrier, 3)

    S, KB = [], []
    for e in range(3):
        s, kb = [ENGINE_BASE[e]], []
        for r in range(3):
            b = _keep_bit(my, MASKS[e][r])
            kb.append(b)
            s.append(s[-1] + b * (ENGINE_ROWS[e] >> (r + 1)))
        S.append(s)
        KB.append(kb)

    scale = sx_ref[0] * sw_ref[0]
    w = w_ref[...].astype(jnp.bfloat16)

    def start_sends(e, r, n_sub):
        H = ENGINE_ROWS[e] >> (r + 1)
        Hs = H // n_sub
        partner = my ^ MASKS[e][r]
        send_base = S[e][r] + (1 - KB[e][r]) * H
        subs = []
        for j in range(n_sub):
            rd = pltpu.make_async_remote_copy(
                src_ref=out_ref.at[pl.ds(send_base + j * Hs, Hs), :],
                dst_ref=bufs[e].at[pl.ds(j * Hs, Hs), :],
                send_sem=rs_send.at[e, r, j],
                recv_sem=rs_recv.at[e, r, j],
                device_id=(partner,),
                device_id_type=_MESH,
            )
            rd.start()
            subs.append(rd)
        return subs

    for r in range(3):
        n_sub = 2 if r < 2 else 1
        rdmas = []
        for e in range(3):
            if r == 0:
                rows = pl.ds(ENGINE_BASE[e], ENGINE_ROWS[e])
                xb = x_ref[rows, :].astype(jnp.bfloat16)
                out_ref[rows, :] = scale * lax.dot_general(
                    xb, w, (((1,), (0,)), ((), ())),
                    preferred_element_type=jnp.float32,
                )
            else:
                pl.semaphore_wait(credits.at[e], 1)
            rdmas.append(start_sends(e, r, n_sub))
        for j in range(n_sub):
            for e in range(3):
                Hs = (ENGINE_ROWS[e] >> (r + 1)) // n_sub
                rdmas[e][j].wait_recv()
                rows = pl.ds(S[e][r + 1] + j * Hs, Hs)
                out_ref[rows, :] += bufs[e][pl.ds(j * Hs, Hs), :]
        for e in range(3):
            for j in range(n_sub):
                rdmas[e][j].wait_send()
            if r < 2:
                pl.semaphore_signal(
                    credits.at[e], inc=1,
                    device_id=(my ^ MASKS[e][r + 1],), device_id_type=_MESH,
                )

    for k in (2, 1, 0):
        rdmas = []
        for e in range(3):
            L = ENGINE_ROWS[e] >> (k + 1)
            rows = pl.ds(S[e][k + 1], L)
            rd = pltpu.make_async_remote_copy(
                src_ref=out_ref.at[rows, :],
                dst_ref=out_ref.at[rows, :],
                send_sem=ag_send.at[e, k],
                recv_sem=ag_recv.at[e, k],
                device_id=(my ^ MASKS[e][k],),
                device_id_type=_MESH,
            )
            rd.start()
            rdmas.append(rd)
        for e in range(3):
            rdmas[e].wait()


def kernel(x, w_mat, scale_x, scale_w):
    return pl.pallas_call(
        _ar_body,
        out_shape=jax.ShapeDtypeStruct((M, N), jnp.float32),
        in_specs=[
            pl.BlockSpec(memory_space=pltpu.VMEM),
            pl.BlockSpec(memory_space=pltpu.VMEM),
            pl.BlockSpec(memory_space=pltpu.SMEM),
            pl.BlockSpec(memory_space=pltpu.SMEM),
        ],
        out_specs=pl.BlockSpec(memory_space=pltpu.VMEM),
        scratch_shapes=[
            pltpu.VMEM((ENGINE_ROWS[0] // 2, N), jnp.float32),
            pltpu.VMEM((ENGINE_ROWS[1] // 2, N), jnp.float32),
            pltpu.VMEM((ENGINE_ROWS[2] // 2, N), jnp.float32),
            pltpu.SemaphoreType.DMA((3, 3, 2)),
            pltpu.SemaphoreType.DMA((3, 3, 2)),
            pltpu.SemaphoreType.DMA((3, 3)),
            pltpu.SemaphoreType.DMA((3, 3)),
            pltpu.SemaphoreType.REGULAR((3,)),
        ],
        compiler_params=pltpu.CompilerParams(
            collective_id=0,
            vmem_limit_bytes=60 * 1024 * 1024,
        ),
    )(x, w_mat, scale_x, scale_w)


# device time: 272965 ns/iter; 2.6515x vs baseline; 1.0115x over previous
import jax
import jax.numpy as jnp
from jax import lax
from jax.experimental import pallas as pl
from jax.experimental.pallas import tpu as pltpu

N_DEV = 8
M, N, K = 4096, 2048, 512
ENGINE_ROWS = (1408, 1344, 1344)
ENGINE_BASE = (0, 1408, 2752)
MASKS = ((1, 3, 4), (3, 4, 1), (4, 1, 3))
_MESH = pl.DeviceIdType.MESH


def _keep_bit(my, m):
    if m == 1:
        return (my ^ (my >> 1)) & 1
    if m == 3:
        return (my >> 1) & 1
    return (my >> 2) & 1


def _ar_body(x_ref, w_ref, sx_ref, sw_ref, out_ref, buf0, buf1, buf2,
             rs_send, rs_recv, ag_send, ag_recv, credits):
    bufs = (buf0, buf1, buf2)
    my = lax.axis_index("i")

    barrier = pltpu.get_barrier_semaphore()
    for m in (1, 3, 4):
        pl.semaphore_signal(
            barrier, inc=1, device_id=(my ^ m,), device_id_type=_MESH,
        )
    pl.semaphore_wait(barrier, 3)

    S, KB = [], []
    for e in range(3):
        s, kb = [ENGINE_BASE[e]], []
        for r in range(3):
            b = _keep_bit(my, MASKS[e][r])
            kb.append(b)
            s.append(s[-1] + b * (ENGINE_ROWS[e] >> (r + 1)))
        S.append(s)
        KB.append(kb)

    scale = sx_ref[0] * sw_ref[0]
    w = w_ref[...].astype(jnp.bfloat16)

    def gemm_block(base, nrows):
        rows = pl.ds(base, nrows)
        xb = x_ref[rows, :].astype(jnp.bfloat16)
        out_ref[rows, :] = scale * lax.dot_general(
            xb, w, (((1,), (0,)), ((), ())),
            preferred_element_type=jnp.float32,
        )

    def gemm_half(e, kept):
        H = ENGINE_ROWS[e] >> 1
        lo = (KB[e][0] == 0) if kept else (KB[e][0] == 1)

        @pl.when(lo)
        def _():
            gemm_block(ENGINE_BASE[e], H)

        @pl.when(jnp.logical_not(lo))
        def _():
            gemm_block(ENGINE_BASE[e] + H, H)

    def start_sends(e, r, n_sub):
        H = ENGINE_ROWS[e] >> (r + 1)
        Hs = H // n_sub
        partner = my ^ MASKS[e][r]
        send_base = S[e][r] + (1 - KB[e][r]) * H
        subs = []
        for j in range(n_sub):
            rd = pltpu.make_async_remote_copy(
                src_ref=out_ref.at[pl.ds(send_base + j * Hs, Hs), :],
                dst_ref=bufs[e].at[pl.ds(j * Hs, Hs), :],
                send_sem=rs_send.at[e, r, j],
                recv_sem=rs_recv.at[e, r, j],
                device_id=(partner,),
                device_id_type=_MESH,
            )
            rd.start()
            subs.append(rd)
        return subs

    for r in range(3):
        n_sub = 2 if r < 2 else 1
        rdmas = []
        for e in range(3):
            if r == 0:
                gemm_half(e, kept=False)
            else:
                pl.semaphore_wait(credits.at[e], 1)
            rdmas.append(start_sends(e, r, n_sub))
        if r == 0:
            for e in range(3):
                gemm_half(e, kept=True)
        for j in range(n_sub):
            for e in range(3):
                Hs = (ENGINE_ROWS[e] >> (r + 1)) // n_sub
                rdmas[e][j].wait_recv()
                rows = pl.ds(S[e][r + 1] + j * Hs, Hs)
                out_ref[rows, :] += bufs[e][pl.ds(j * Hs, Hs), :]
        for e in range(3):
            for j in range(n_sub):
                rdmas[e][j].wait_send()
            if r < 2:
                pl.semaphore_signal(
                    credits.at[e], inc=1,
                    device_id=(my ^ MASKS[e][r + 1],), device_id_type=_MESH,
                )

    for k in (2, 1, 0):
        rdmas = []
        for e in range(3):
            L = ENGINE_ROWS[e] >> (k + 1)
            rows = pl.ds(S[e][k + 1], L)
            rd = pltpu.make_async_remote_copy(
                src_ref=out_ref.at[rows, :],
                dst_ref=out_ref.at[rows, :],
                send_sem=ag_send.at[e, k],
                recv_sem=ag_recv.at[e, k],
                device_id=(my ^ MASKS[e][k],),
                device_id_type=_MESH,
            )
            rd.start()
            rdmas.append(rd)
        for e in range(3):
            rdmas[e].wait()


def kernel(x, w_mat, scale_x, scale_w):
    return pl.pallas_call(
        _ar_body,
        out_shape=jax.ShapeDtypeStruct((M, N), jnp.float32),
        in_specs=[
            pl.BlockSpec(memory_space=pltpu.VMEM),
            pl.BlockSpec(memory_space=pltpu.VMEM),
            pl.BlockSpec(memory_space=pltpu.SMEM),
            pl.BlockSpec(memory_space=pltpu.SMEM),
        ],
        out_specs=pl.BlockSpec(memory_space=pltpu.VMEM),
        scratch_shapes=[
            pltpu.VMEM((ENGINE_ROWS[0] // 2, N), jnp.float32),
            pltpu.VMEM((ENGINE_ROWS[1] // 2, N), jnp.float32),
            pltpu.VMEM((ENGINE_ROWS[2] // 2, N), jnp.float32),
            pltpu.SemaphoreType.DMA((3, 3, 2)),
            pltpu.SemaphoreType.DMA((3, 3, 2)),
            pltpu.SemaphoreType.DMA((3, 3)),
            pltpu.SemaphoreType.DMA((3, 3)),
            pltpu.SemaphoreType.REGULAR((3,)),
        ],
        compiler_params=pltpu.CompilerParams(
            collective_id=0,
            vmem_limit_bytes=60 * 1024 * 1024,
        ),
    )(x, w_mat, scale_x, scale_w)
